# baseline (speedup 1.0000x reference)
import functools

import jax
import jax.numpy as jnp
from jax import lax
from jax.experimental import pallas as pl
from jax.experimental.pallas import tpu as pltpu
from jax.experimental.pallas import tpu_sc as plsc

BATCH = 16384
EMBED_K = 32
NC = 2
NS = 16
NW = NC * NS
BPW = BATCH // NW
IDX_MINOR = 128
NGRP = BPW // IDX_MINOR


def _mf_body(uidx_hbm, iidx_hbm, user_table, item_table,
             out_hbm, uemb_hbm, iemb_hbm,
             uidx_v, iidx_v, urows_v, irows_v, out_v, sem):
    wid = lax.axis_index("s") * NC + lax.axis_index("c")
    base = wid * BPW

    pltpu.sync_copy(uidx_hbm.at[wid], uidx_v)
    pltpu.sync_copy(iidx_hbm.at[wid], iidx_v)

    copies = []
    for j in range(NGRP):
        copies.append(pltpu.async_copy(
            user_table.at[uidx_v.at[j]],
            urows_v.at[pl.ds(j * IDX_MINOR, IDX_MINOR)], sem))
        copies.append(pltpu.async_copy(
            item_table.at[iidx_v.at[j]],
            irows_v.at[pl.ds(j * IDX_MINOR, IDX_MINOR)], sem))
    for c in copies:
        c.wait()

    last_lane = lax.iota(jnp.int32, 16) == 15

    def dot_row(i, carry):
        u0 = urows_v[i, pl.ds(0, 16)]
        u1 = urows_v[i, pl.ds(16, 16)]
        v0 = irows_v[i, pl.ds(0, 16)]
        v1 = irows_v[i, pl.ds(16, 16)]
        w = u0 * v0 + u1 * v1
        s = plsc.cumsum(w)
        plsc.store_scatter(out_v, [jnp.full((16,), i, jnp.int32)], s,
                           mask=last_lane)
        return carry

    lax.fori_loop(0, BPW, dot_row, 0, unroll=8)

    pltpu.sync_copy(urows_v, uemb_hbm.at[pl.ds(base, BPW)])
    pltpu.sync_copy(irows_v, iemb_hbm.at[pl.ds(base, BPW)])
    pltpu.sync_copy(out_v, out_hbm.at[pl.ds(base, BPW)])


@functools.partial(jax.jit, static_argnames=())
def _mf(uidx, iidx, user_table, item_table):
    kern = pl.kernel(
        _mf_body,
        out_type=[
            jax.ShapeDtypeStruct((BATCH,), jnp.float32),
            jax.ShapeDtypeStruct((BATCH, EMBED_K), jnp.float32),
            jax.ShapeDtypeStruct((BATCH, EMBED_K), jnp.float32),
        ],
        mesh=plsc.VectorSubcoreMesh(core_axis_name="c", subcore_axis_name="s"),
        scratch_types=[
            pltpu.VMEM((NGRP, IDX_MINOR), jnp.int32),
            pltpu.VMEM((NGRP, IDX_MINOR), jnp.int32),
            pltpu.VMEM((BPW, EMBED_K), jnp.float32),
            pltpu.VMEM((BPW, EMBED_K), jnp.float32),
            pltpu.VMEM((BPW,), jnp.float32),
            pltpu.SemaphoreType.DMA,
        ],
        compiler_params=pltpu.CompilerParams(
            needs_layout_passes=False, use_tc_tiling_on_sc=False),
    )
    return kern(uidx, iidx, user_table, item_table)


def kernel(x, user_table, item_table):
    xi = x.astype(jnp.int32)
    uidx = xi[:, 0].reshape(NW, NGRP, IDX_MINOR)
    iidx = xi[:, 1].reshape(NW, NGRP, IDX_MINOR)
    out, uemb, iemb = _mf(uidx, iidx, user_table, item_table)
    return (out[:, None], uemb, iemb)
